# Initial kernel scaffold; baseline (speedup 1.0000x reference)
import jax
import jax.numpy as jnp
from jax.experimental import pallas as pl


def kernel(location, loc_to_group, thc_params, thg_params, tbg_params):
    raise NotImplementedError("write your pallas kernel here")


# speedup vs baseline: 7.6872x; 7.6872x over previous
import jax
import jax.numpy as jnp
from jax import lax
from jax.experimental import pallas as pl
from jax.experimental.pallas import tpu as pltpu
from jax.experimental.pallas import tpu_sc as plsc

_B = 16384
_L = 16
_NC = 2
_NS = 16
_NW = _NC * _NS
_BPW = _B // _NW

_L2G_PAD = 112
_PRM_PAD = 32


def _sc_body(loc_hbm, l2g_hbm, thc_hbm, thg_hbm, tbg_hbm,
             thc_out, thg_out, tbg_out,
             loc_v, l2g_v, thc_v, thg_v, tbg_v,
             othc_v, othg_v, otbg_v):
    wid = lax.axis_index("s") * _NC + lax.axis_index("c")
    base = wid * _BPW
    pltpu.sync_copy(l2g_hbm, l2g_v)
    pltpu.sync_copy(thc_hbm, thc_v)
    pltpu.sync_copy(thg_hbm, thg_v)
    pltpu.sync_copy(tbg_hbm, tbg_v)
    pltpu.sync_copy(loc_hbm.at[pl.ds(base, _BPW)], loc_v)
    for i in range(_BPW // _L):
        sl = pl.ds(i * _L, _L)
        lv = loc_v[sl]
        g = plsc.load_gather(l2g_v, [lv])
        a = plsc.load_gather(thc_v, [g])
        b = plsc.load_gather(thg_v, [g])
        c = plsc.load_gather(tbg_v, [g])
        othc_v[sl] = jnp.maximum(a, 0.0)
        othg_v[sl] = jnp.maximum(b, 0.0)
        otbg_v[sl] = jnp.maximum(c, 0.0) * 20.0
    pltpu.sync_copy(othc_v, thc_out.at[pl.ds(base, _BPW)])
    pltpu.sync_copy(othg_v, thg_out.at[pl.ds(base, _BPW)])
    pltpu.sync_copy(otbg_v, tbg_out.at[pl.ds(base, _BPW)])


def kernel(location, loc_to_group, thc_params, thg_params, tbg_params):
    l2g = jnp.pad(loc_to_group, (0, _L2G_PAD - loc_to_group.shape[0]))
    thc = jnp.pad(thc_params, (0, _PRM_PAD - thc_params.shape[0]))
    thg = jnp.pad(thg_params, (0, _PRM_PAD - thg_params.shape[0]))
    tbg = jnp.pad(tbg_params, (0, _PRM_PAD - tbg_params.shape[0]))
    f = pl.kernel(
        _sc_body,
        out_type=[jax.ShapeDtypeStruct((_B,), jnp.float32)] * 3,
        mesh=plsc.VectorSubcoreMesh(core_axis_name="c", subcore_axis_name="s"),
        compiler_params=pltpu.CompilerParams(needs_layout_passes=False),
        scratch_types=[
            pltpu.VMEM((_BPW,), jnp.int32),
            pltpu.VMEM((_L2G_PAD,), jnp.int32),
            pltpu.VMEM((_PRM_PAD,), jnp.float32),
            pltpu.VMEM((_PRM_PAD,), jnp.float32),
            pltpu.VMEM((_PRM_PAD,), jnp.float32),
            pltpu.VMEM((_BPW,), jnp.float32),
            pltpu.VMEM((_BPW,), jnp.float32),
            pltpu.VMEM((_BPW,), jnp.float32),
        ],
    )
    thc_o, thg_o, tbg_o = f(location, l2g, thc, thg, tbg)
    return (thc_o.reshape(-1, 1), thg_o.reshape(-1, 1), tbg_o.reshape(-1, 1))


# speedup vs baseline: 8.4041x; 1.0933x over previous
import jax
import jax.numpy as jnp
from jax import lax
from jax.experimental import pallas as pl
from jax.experimental.pallas import tpu as pltpu
from jax.experimental.pallas import tpu_sc as plsc

_B = 16384
_L = 16
_NC = 2
_NS = 16
_NW = _NC * _NS
_BPW = _B // _NW

_N_LOC = 100
_N_GRP = 26


def _sc_body(loc_hbm, l2g_hbm, thc_hbm, thg_hbm, tbg_hbm,
             thc_out, thg_out, tbg_out,
             loc_v, l2g_v, thc_v, thg_v, tbg_v,
             othc_v, othg_v, otbg_v, sem):
    wid = lax.axis_index("s") * _NC + lax.axis_index("c")
    base = wid * _BPW
    cps = [
        pltpu.async_copy(loc_hbm.at[pl.ds(base, _BPW)], loc_v, sem),
        pltpu.async_copy(l2g_hbm, l2g_v, sem),
        pltpu.async_copy(thc_hbm, thc_v, sem),
        pltpu.async_copy(thg_hbm, thg_v, sem),
        pltpu.async_copy(tbg_hbm, tbg_v, sem),
    ]
    for cp in cps:
        cp.wait()
    for i in range(_BPW // _L):
        sl = pl.ds(i * _L, _L)
        lv = loc_v[sl]
        g = plsc.load_gather(l2g_v, [lv])
        a = plsc.load_gather(thc_v, [g])
        b = plsc.load_gather(thg_v, [g])
        c = plsc.load_gather(tbg_v, [g])
        othc_v[sl] = jnp.maximum(a, 0.0)
        othg_v[sl] = jnp.maximum(b, 0.0)
        otbg_v[sl] = jnp.maximum(c, 0.0) * 20.0
    outs = [
        pltpu.async_copy(othc_v, thc_out.at[pl.ds(base, _BPW)], sem),
        pltpu.async_copy(othg_v, thg_out.at[pl.ds(base, _BPW)], sem),
        pltpu.async_copy(otbg_v, tbg_out.at[pl.ds(base, _BPW)], sem),
    ]
    for cp in outs:
        cp.wait()


def kernel(location, loc_to_group, thc_params, thg_params, tbg_params):
    f = pl.kernel(
        _sc_body,
        out_type=[jax.ShapeDtypeStruct((_B,), jnp.float32)] * 3,
        mesh=plsc.VectorSubcoreMesh(core_axis_name="c", subcore_axis_name="s"),
        compiler_params=pltpu.CompilerParams(needs_layout_passes=False),
        scratch_types=[
            pltpu.VMEM((_BPW,), jnp.int32),
            pltpu.VMEM((_N_LOC,), jnp.int32),
            pltpu.VMEM((_N_GRP,), jnp.float32),
            pltpu.VMEM((_N_GRP,), jnp.float32),
            pltpu.VMEM((_N_GRP,), jnp.float32),
            pltpu.VMEM((_BPW,), jnp.float32),
            pltpu.VMEM((_BPW,), jnp.float32),
            pltpu.VMEM((_BPW,), jnp.float32),
            pltpu.SemaphoreType.DMA,
        ],
    )
    thc_o, thg_o, tbg_o = f(location, loc_to_group, thc_params, thg_params,
                            tbg_params)
    return (thc_o.reshape(-1, 1), thg_o.reshape(-1, 1), tbg_o.reshape(-1, 1))


# speedup vs baseline: 8.6578x; 1.0302x over previous
import jax
import jax.numpy as jnp
from jax import lax
from jax.experimental import pallas as pl
from jax.experimental.pallas import tpu as pltpu
from jax.experimental.pallas import tpu_sc as plsc

_B = 16384
_L = 16
_NC = 2
_NS = 16
_NW = _NC * _NS
_BPW = _B // _NW

_N_LOC = 100
_N_GRP = 26


def _sc_body(loc_hbm, l2g_hbm, thc_hbm, thg_hbm, tbg_hbm,
             thc_out, thg_out, tbg_out,
             loc_v, l2g_v, thc_v, thg_v, tbg_v,
             othc_v, othg_v, otbg_v, sem):
    wid = lax.axis_index("s") * _NC + lax.axis_index("c")
    base = wid * _BPW
    cps = [
        pltpu.async_copy(loc_hbm.at[pl.ds(base, _BPW)], loc_v, sem),
        pltpu.async_copy(l2g_hbm, l2g_v, sem),
        pltpu.async_copy(thc_hbm, thc_v, sem),
        pltpu.async_copy(thg_hbm, thg_v, sem),
        pltpu.async_copy(tbg_hbm, tbg_v, sem),
    ]
    for cp in cps:
        cp.wait()
    @plsc.parallel_loop(0, _BPW // _L, unroll=4)
    def _(i):
        sl = pl.ds(i * _L, _L)
        lv = loc_v[sl]
        g = plsc.load_gather(l2g_v, [lv])
        a = plsc.load_gather(thc_v, [g])
        b = plsc.load_gather(thg_v, [g])
        c = plsc.load_gather(tbg_v, [g])
        othc_v[sl] = jnp.maximum(a, 0.0)
        othg_v[sl] = jnp.maximum(b, 0.0)
        otbg_v[sl] = jnp.maximum(c, 0.0) * 20.0
    outs = [
        pltpu.async_copy(othc_v, thc_out.at[pl.ds(base, _BPW)], sem),
        pltpu.async_copy(othg_v, thg_out.at[pl.ds(base, _BPW)], sem),
        pltpu.async_copy(otbg_v, tbg_out.at[pl.ds(base, _BPW)], sem),
    ]
    for cp in outs:
        cp.wait()


def kernel(location, loc_to_group, thc_params, thg_params, tbg_params):
    f = pl.kernel(
        _sc_body,
        out_type=[jax.ShapeDtypeStruct((_B,), jnp.float32)] * 3,
        mesh=plsc.VectorSubcoreMesh(core_axis_name="c", subcore_axis_name="s"),
        compiler_params=pltpu.CompilerParams(needs_layout_passes=False),
        scratch_types=[
            pltpu.VMEM((_BPW,), jnp.int32),
            pltpu.VMEM((_N_LOC,), jnp.int32),
            pltpu.VMEM((_N_GRP,), jnp.float32),
            pltpu.VMEM((_N_GRP,), jnp.float32),
            pltpu.VMEM((_N_GRP,), jnp.float32),
            pltpu.VMEM((_BPW,), jnp.float32),
            pltpu.VMEM((_BPW,), jnp.float32),
            pltpu.VMEM((_BPW,), jnp.float32),
            pltpu.SemaphoreType.DMA,
        ],
    )
    thc_o, thg_o, tbg_o = f(location, loc_to_group, thc_params, thg_params,
                            tbg_params)
    return (thc_o.reshape(-1, 1), thg_o.reshape(-1, 1), tbg_o.reshape(-1, 1))


# speedup vs baseline: 9.3364x; 1.0784x over previous
import jax
import jax.numpy as jnp
from jax import lax
from jax.experimental import pallas as pl
from jax.experimental.pallas import tpu as pltpu
from jax.experimental.pallas import tpu_sc as plsc

_B = 16384
_L = 16
_NC = 1
_NS = 16
_NW = _NC * _NS
_BPW = _B // _NW

_N_LOC = 100
_N_GRP = 26


def _sc_body(loc_hbm, l2g_hbm, thc_hbm, thg_hbm, tbg_hbm,
             thc_out, thg_out, tbg_out,
             loc_v, l2g_v, thc_v, thg_v, tbg_v,
             othc_v, othg_v, otbg_v, sem):
    wid = lax.axis_index("s") * _NC + lax.axis_index("c")
    base = wid * _BPW
    cps = [
        pltpu.async_copy(loc_hbm.at[pl.ds(base, _BPW)], loc_v, sem),
        pltpu.async_copy(l2g_hbm, l2g_v, sem),
        pltpu.async_copy(thc_hbm, thc_v, sem),
        pltpu.async_copy(thg_hbm, thg_v, sem),
        pltpu.async_copy(tbg_hbm, tbg_v, sem),
    ]
    for cp in cps:
        cp.wait()
    @plsc.parallel_loop(0, _BPW // _L, unroll=4)
    def _(i):
        sl = pl.ds(i * _L, _L)
        lv = loc_v[sl]
        g = plsc.load_gather(l2g_v, [lv])
        a = plsc.load_gather(thc_v, [g])
        b = plsc.load_gather(thg_v, [g])
        c = plsc.load_gather(tbg_v, [g])
        othc_v[sl] = jnp.maximum(a, 0.0)
        othg_v[sl] = jnp.maximum(b, 0.0)
        otbg_v[sl] = jnp.maximum(c, 0.0) * 20.0
    outs = [
        pltpu.async_copy(othc_v, thc_out.at[pl.ds(base, _BPW)], sem),
        pltpu.async_copy(othg_v, thg_out.at[pl.ds(base, _BPW)], sem),
        pltpu.async_copy(otbg_v, tbg_out.at[pl.ds(base, _BPW)], sem),
    ]
    for cp in outs:
        cp.wait()


def kernel(location, loc_to_group, thc_params, thg_params, tbg_params):
    f = pl.kernel(
        _sc_body,
        out_type=[jax.ShapeDtypeStruct((_B,), jnp.float32)] * 3,
        mesh=plsc.VectorSubcoreMesh(core_axis_name="c", subcore_axis_name="s",
                                    num_cores=_NC),
        compiler_params=pltpu.CompilerParams(needs_layout_passes=False),
        scratch_types=[
            pltpu.VMEM((_BPW,), jnp.int32),
            pltpu.VMEM((_N_LOC,), jnp.int32),
            pltpu.VMEM((_N_GRP,), jnp.float32),
            pltpu.VMEM((_N_GRP,), jnp.float32),
            pltpu.VMEM((_N_GRP,), jnp.float32),
            pltpu.VMEM((_BPW,), jnp.float32),
            pltpu.VMEM((_BPW,), jnp.float32),
            pltpu.VMEM((_BPW,), jnp.float32),
            pltpu.SemaphoreType.DMA,
        ],
    )
    thc_o, thg_o, tbg_o = f(location, loc_to_group, thc_params, thg_params,
                            tbg_params)
    return (thc_o.reshape(-1, 1), thg_o.reshape(-1, 1), tbg_o.reshape(-1, 1))
